# baseline (speedup 1.0000x reference)
import functools

import jax
import jax.numpy as jnp
from jax import lax
from jax.experimental import pallas as pl
from jax.experimental.pallas import tpu as pltpu
from jax.experimental.pallas import tpu_sc as plsc

N_NODES = 10000
N_EDGES = 320000
DIM = 128
LAMBDA_WEIGHT = 0.1
L = 16
NW = 32
CHUNK = 2 * N_EDGES // NW

_mesh = plsc.VectorSubcoreMesh(core_axis_name="c", subcore_axis_name="s")


@functools.partial(
    pl.kernel,
    out_type=jax.ShapeDtypeStruct((NW, N_NODES), jnp.int32),
    mesh=_mesh,
    scratch_types=[
        pltpu.VMEM((CHUNK,), jnp.int32),
        pltpu.VMEM((N_NODES,), jnp.int32),
    ],
    compiler_params=pltpu.CompilerParams(needs_layout_passes=False),
)
def _hist_kernel(idx_hbm, out_hbm, idx_v, hist_v):
    c = lax.axis_index("c")
    s = lax.axis_index("s")
    wid = s * 2 + c
    base = wid * CHUNK
    pltpu.sync_copy(idx_hbm.at[pl.ds(base, CHUNK)], idx_v)

    zeros = jnp.zeros((L,), jnp.int32)

    def zero_body(i, carry):
        hist_v[pl.ds(i * L, L)] = zeros
        return carry

    lax.fori_loop(0, N_NODES // L, zero_body, 0, unroll=16)

    ones = jnp.ones((L,), jnp.int32)

    def body(i, carry):
        idx = idx_v[pl.ds(i * L, L)]
        plsc.addupdate_scatter(hist_v, [idx], ones)
        return carry

    lax.fori_loop(0, CHUNK // L, body, 0, unroll=16)

    pltpu.sync_copy(hist_v, out_hbm.at[wid])


def _reduce_body(emb_ref, parts_ref, out_ref):
    parts = parts_ref[...].astype(jnp.float32)
    cnt_src = jnp.sum(parts[: NW // 2], axis=0, keepdims=True)
    cnt_dst = jnp.sum(parts[NW // 2 :], axis=0, keepdims=True)
    emb = emb_ref[...]
    dot = functools.partial(
        lax.dot_general,
        precision=lax.Precision.HIGHEST,
        preferred_element_type=jnp.float32,
    )
    S = dot(cnt_dst, emb, (((1,), (0,)), ((), ())))
    T = dot(cnt_src, emb, (((1,), (0,)), ((), ())))
    ssq = jnp.sum(emb * emb, axis=1, keepdims=True)
    R = dot(cnt_src, ssq, (((1,), (0,)), ((), ())))
    TS = jnp.sum(T * S)
    SS = jnp.sum(S * S)
    loss = (LAMBDA_WEIGHT / (N_EDGES * DIM)) * (
        R[0, 0] - 2.0 * TS + N_EDGES * SS
    )
    out_ref[0, 0] = loss


def kernel(embeddings, edge_index):
    idx = edge_index.reshape(-1).astype(jnp.int32)
    parts = _hist_kernel(idx)
    loss = pl.pallas_call(
        _reduce_body,
        out_shape=jax.ShapeDtypeStruct((1, 1), jnp.float32),
        out_specs=pl.BlockSpec(memory_space=pltpu.SMEM),
    )(embeddings, parts)
    return loss[0, 0]


# speedup vs baseline: 1.1143x; 1.1143x over previous
import functools

import jax
import jax.numpy as jnp
from jax import lax
from jax.experimental import pallas as pl
from jax.experimental.pallas import tpu as pltpu
from jax.experimental.pallas import tpu_sc as plsc

N_NODES = 10000
N_EDGES = 320000
DIM = 128
LAMBDA_WEIGHT = 0.1
L = 16
NW = 32
C_BLK = 2560
N_BLK = N_EDGES // C_BLK
FULL_ROUNDS = N_BLK // NW
TAIL_WORKERS = N_BLK - FULL_ROUNDS * NW
ITERS = C_BLK // L

_mesh = plsc.VectorSubcoreMesh(core_axis_name="c", subcore_axis_name="s")


@functools.partial(
    pl.kernel,
    out_type=jax.ShapeDtypeStruct((2 * NW, N_NODES), jnp.int32),
    mesh=_mesh,
    scratch_types=[
        pltpu.VMEM((FULL_ROUNDS + 1, 2, C_BLK), jnp.int32),
        pltpu.VMEM((N_NODES,), jnp.int32),
        pltpu.VMEM((N_NODES,), jnp.int32),
        pltpu.SemaphoreType.DMA,
    ],
    compiler_params=pltpu.CompilerParams(needs_layout_passes=False),
)
def _hist_kernel(edge_hbm, out_hbm, slab_v, hsrc_v, hdst_v, sem):
    c = lax.axis_index("c")
    s = lax.axis_index("s")
    wid = s * 2 + c

    copies = []
    for b in range(FULL_ROUNDS):
        blk = wid + b * NW
        copies.append(
            pltpu.async_copy(
                edge_hbm.at[:, pl.ds(blk * C_BLK, C_BLK)], slab_v.at[b], sem
            )
        )
    has_tail = wid < TAIL_WORKERS

    @pl.when(has_tail)
    def _():
        blk = wid + FULL_ROUNDS * NW
        pltpu.async_copy(
            edge_hbm.at[:, pl.ds(blk * C_BLK, C_BLK)],
            slab_v.at[FULL_ROUNDS],
            sem,
        )

    zeros = jnp.zeros((L,), jnp.int32)

    def zero_body(i, carry):
        hsrc_v[pl.ds(i * L, L)] = zeros
        hdst_v[pl.ds(i * L, L)] = zeros
        return carry

    lax.fori_loop(0, N_NODES // L, zero_body, 0, unroll=8)

    for cp in copies:
        cp.wait()

    @pl.when(has_tail)
    def _():
        pltpu.make_async_copy(
            edge_hbm.at[:, pl.ds(0, C_BLK)], slab_v.at[FULL_ROUNDS], sem
        ).wait()

    ones = jnp.ones((L,), jnp.int32)

    def scatter_block(b):
        def body(i, carry):
            src_idx = slab_v[b, 0, pl.ds(i * L, L)]
            plsc.addupdate_scatter(hsrc_v, [src_idx], ones)
            dst_idx = slab_v[b, 1, pl.ds(i * L, L)]
            plsc.addupdate_scatter(hdst_v, [dst_idx], ones)
            return carry

        lax.fori_loop(0, ITERS, body, 0, unroll=8)

    for b in range(FULL_ROUNDS):
        scatter_block(b)

    @pl.when(has_tail)
    def _():
        scatter_block(FULL_ROUNDS)

    pltpu.sync_copy(hsrc_v, out_hbm.at[wid])
    pltpu.sync_copy(hdst_v, out_hbm.at[NW + wid])


def _reduce_body(emb_ref, parts_ref, out_ref):
    parts = parts_ref[...].astype(jnp.float32)
    emb = emb_ref[...]
    M = lax.dot_general(
        parts, emb, (((1,), (0,)), ((), ())),
        precision=lax.Precision.HIGHEST,
        preferred_element_type=jnp.float32,
    )
    ssq = jnp.sum(emb * emb, axis=1, keepdims=True)
    r = lax.dot_general(
        parts[:NW], ssq, (((1,), (0,)), ((), ())),
        precision=lax.Precision.DEFAULT,
        preferred_element_type=jnp.float32,
    )
    T = jnp.sum(M[:NW], axis=0, keepdims=True)
    S = jnp.sum(M[NW:], axis=0, keepdims=True)
    R = jnp.sum(r)
    TS = jnp.sum(T * S)
    SS = jnp.sum(S * S)
    loss = (LAMBDA_WEIGHT / (N_EDGES * DIM)) * (
        R - 2.0 * TS + N_EDGES * SS
    )
    out_ref[0, 0] = loss


def kernel(embeddings, edge_index):
    idx = edge_index.astype(jnp.int32)
    parts = _hist_kernel(idx)
    loss = pl.pallas_call(
        _reduce_body,
        out_shape=jax.ShapeDtypeStruct((1, 1), jnp.float32),
        out_specs=pl.BlockSpec(memory_space=pltpu.SMEM),
    )(embeddings, parts)
    return loss[0, 0]
